# baseline (speedup 1.0000x reference)
import jax
import jax.numpy as jnp
from jax import lax
from jax.experimental import pallas as pl
from jax.experimental.pallas import tpu as pltpu
from jax.experimental.pallas import tpu_sc as plsc

_DIM = 32
_B = 16384
_L = 50
_NC = 2
_NS = 16
_NW = _NC * _NS
_BPW = _B // _NW
_NV = _BPW // 16


def _body(x_hbm, m_hbm, w_hbm, out_hbm, xb, mb, acc, sem0, sem):
    c = lax.axis_index("c")
    s = lax.axis_index("s")
    wid = s * _NC + c
    base = wid * _BPW

    pltpu.sync_copy(x_hbm.at[:, pl.ds(base, _BPW)], xb)
    pltpu.sync_copy(m_hbm.at[:, pl.ds(base, _BPW)], mb)

    def sel_row(l, carry):
        for i in range(_NV):
            xv = xb[l, pl.ds(i * 16, 16)]
            mv = mb[l, pl.ds(i * 16, 16)]
            xb[l, pl.ds(i * 16, 16)] = jnp.where(mv > 0, xv, 0)
        return carry

    lax.fori_loop(0, _L, sel_row, 0)

    pltpu.async_copy(w_hbm.at[xb.at[0]], acc, sem0).wait()

    for l in range(1, _L):
        pltpu.async_copy(w_hbm.at[xb.at[l]], acc, sem, add=True)
    for l in range(1, _L):
        pltpu.make_async_copy(w_hbm.at[xb.at[0]], acc, sem).wait()

    pltpu.sync_copy(acc, out_hbm.at[pl.ds(base, _BPW)])


def kernel(x, mask, W):
    xt = x.T
    mt = mask[:, :, 0].astype(jnp.int32).T
    mesh = plsc.VectorSubcoreMesh(
        core_axis_name="c", subcore_axis_name="s",
        num_cores=_NC, num_subcores=_NS,
    )
    k = pl.kernel(
        _body,
        out_type=jax.ShapeDtypeStruct((_B, _DIM), jnp.float32),
        mesh=mesh,
        compiler_params=pltpu.CompilerParams(use_tc_tiling_on_sc=False),
        scratch_types=[
            pltpu.VMEM((_L, _BPW), jnp.int32),
            pltpu.VMEM((_L, _BPW), jnp.int32),
            pltpu.VMEM((_BPW, _DIM), jnp.float32),
            pltpu.SemaphoreType.DMA,
            pltpu.SemaphoreType.DMA,
        ],
    )
    return k(xt, mt, W)


# speedup vs baseline: 4.9131x; 4.9131x over previous
import jax
import jax.numpy as jnp
from jax import lax
from jax.experimental import pallas as pl
from jax.experimental.pallas import tpu as pltpu
from jax.experimental.pallas import tpu_sc as plsc

_DIM = 32
_VOCAB = 1000000
_NPAD = 2048
_B = 16384
_L = 50
_NC = 2
_NS = 16
_NW = _NC * _NS
_BPW = _B // _NW
_NV = _BPW // 16


def _body(x_hbm, m_hbm, w_hbm, out_hbm, xb, mb, acc, sem0, sem):
    c = lax.axis_index("c")
    s = lax.axis_index("s")
    wid = s * _NC + c
    base = wid * _BPW

    pltpu.sync_copy(x_hbm.at[:, pl.ds(base, _BPW)], xb)
    pltpu.sync_copy(m_hbm.at[:, pl.ds(base, _BPW)], mb)

    def sel_row(l, carry):
        for i in range(_NV):
            xv = xb[l, pl.ds(i * 16, 16)]
            mv = mb[l, pl.ds(i * 16, 16)]
            xb[l, pl.ds(i * 16, 16)] = jnp.where(
                mv > 0, xv, _VOCAB + (xv & (_NPAD - 1)))
        return carry

    lax.fori_loop(0, _L, sel_row, 0)

    pltpu.async_copy(w_hbm.at[xb.at[0]], acc, sem0).wait()

    for l in range(1, _L):
        pltpu.async_copy(w_hbm.at[xb.at[l]], acc, sem, add=True)
    for l in range(1, _L):
        pltpu.make_async_copy(w_hbm.at[xb.at[0]], acc, sem).wait()

    pltpu.sync_copy(acc, out_hbm.at[pl.ds(base, _BPW)])


def kernel(x, mask, W):
    xt = x.T
    mt = mask[:, :, 0].astype(jnp.int32).T
    wz = jnp.concatenate(
        [W, jnp.zeros((_NPAD, _DIM), jnp.float32)], axis=0)
    mesh = plsc.VectorSubcoreMesh(
        core_axis_name="c", subcore_axis_name="s",
        num_cores=_NC, num_subcores=_NS,
    )
    k = pl.kernel(
        _body,
        out_type=jax.ShapeDtypeStruct((_B, _DIM), jnp.float32),
        mesh=mesh,
        compiler_params=pltpu.CompilerParams(use_tc_tiling_on_sc=False),
        scratch_types=[
            pltpu.VMEM((_L, _BPW), jnp.int32),
            pltpu.VMEM((_L, _BPW), jnp.int32),
            pltpu.VMEM((_BPW, _DIM), jnp.float32),
            pltpu.SemaphoreType.DMA,
            pltpu.SemaphoreType.DMA,
        ],
    )
    return k(xt, mt, wz)
